# Initial kernel scaffold; baseline (speedup 1.0000x reference)
import jax
import jax.numpy as jnp
from jax.experimental import pallas as pl


def kernel(outputs, tokens, tokens_lens):
    raise NotImplementedError("write your pallas kernel here")


# speedup vs baseline: 1.9915x; 1.9915x over previous
import jax
import jax.numpy as jnp
from jax import lax
from jax.experimental import pallas as pl
from jax.experimental.pallas import tpu as pltpu

_TB = 64


def _acc_kernel(lens_ref, x_ref, tgt_ref, out_ref, c_ref, m_ref):
    i = pl.program_id(0)
    nsteps = pl.num_programs(0)
    x = x_ref[...]
    TB, B, V = x.shape
    rowmax = jnp.max(x, axis=-1, keepdims=True)
    idx = lax.broadcasted_iota(jnp.int32, x.shape, 2)
    pred = jnp.min(jnp.where(x == rowmax, idx, V), axis=-1)
    tgt = tgt_ref[0]
    t_idx = lax.broadcasted_iota(jnp.int32, (TB, B), 0) + i * TB
    b_idx = lax.broadcasted_iota(jnp.int32, (TB, B), 1)
    lens_v = jnp.zeros((TB, B), jnp.int32)
    for b in range(B):
        lens_v = jnp.where(b_idx == b, lens_ref[b] + 1, lens_v)
    mask = t_idx < lens_v
    c_part = jnp.sum(jnp.where(mask & (pred == tgt), 1.0, 0.0))
    m_part = jnp.sum(jnp.where(mask, 1.0, 0.0))

    @pl.when(i == 0)
    def _init():
        c_ref[0] = 0.0
        m_ref[0] = 0.0

    c_ref[0] += c_part
    m_ref[0] += m_part

    @pl.when(i == nsteps - 1)
    def _fin():
        out_ref[0] = c_ref[0] / m_ref[0]


def kernel(outputs, tokens, tokens_lens):
    T, B, V = outputs.shape
    tgt = jnp.concatenate([tokens[1:], tokens[-1:]], axis=0)
    ntb = T // _TB
    tgt3 = tgt.reshape(ntb, _TB, B)
    grid_spec = pltpu.PrefetchScalarGridSpec(
        num_scalar_prefetch=1,
        grid=(ntb,),
        in_specs=[
            pl.BlockSpec((_TB, B, V), lambda i, lens: (i, 0, 0)),
            pl.BlockSpec((1, _TB, B), lambda i, lens: (i, 0, 0)),
        ],
        out_specs=pl.BlockSpec(memory_space=pltpu.SMEM),
        scratch_shapes=[
            pltpu.SMEM((1,), jnp.float32),
            pltpu.SMEM((1,), jnp.float32),
        ],
    )
    acc = pl.pallas_call(
        _acc_kernel,
        grid_spec=grid_spec,
        out_shape=jax.ShapeDtypeStruct((1,), jnp.float32),
        compiler_params=pltpu.CompilerParams(
            dimension_semantics=("arbitrary",),
        ),
    )(tokens_lens, outputs, tgt3)
    return acc[0]


# speedup vs baseline: 2.3646x; 1.1874x over previous
import jax
import jax.numpy as jnp
from jax import lax
from jax.experimental import pallas as pl
from jax.experimental.pallas import tpu as pltpu

_TB = 128


def _acc_kernel(lens_ref, x_ref, tgt_ref, out_ref, c_ref, m_ref):
    i = pl.program_id(0)
    nsteps = pl.num_programs(0)
    x = x_ref[...]
    TB, B, V = x.shape
    rowmax = jnp.max(x, axis=-1, keepdims=True)
    idx = lax.broadcasted_iota(jnp.int32, x.shape, 2)
    pred = jnp.min(jnp.where(x == rowmax, idx, V), axis=-1)
    tgt = tgt_ref[0]
    t_idx = lax.broadcasted_iota(jnp.int32, (TB, B), 0) + i * TB
    b_idx = lax.broadcasted_iota(jnp.int32, (TB, B), 1)
    lens_v = jnp.zeros((TB, B), jnp.int32)
    for b in range(B):
        lens_v = jnp.where(b_idx == b, lens_ref[b] + 1, lens_v)
    mask = t_idx < lens_v
    c_part = jnp.sum(jnp.where(mask & (pred == tgt), 1.0, 0.0))
    m_part = jnp.sum(jnp.where(mask, 1.0, 0.0))

    @pl.when(i == 0)
    def _init():
        c_ref[0] = 0.0
        m_ref[0] = 0.0

    c_ref[0] += c_part
    m_ref[0] += m_part

    @pl.when(i == nsteps - 1)
    def _fin():
        out_ref[0] = c_ref[0] / m_ref[0]


def kernel(outputs, tokens, tokens_lens):
    T, B, V = outputs.shape
    tgt = jnp.concatenate([tokens[1:], tokens[-1:]], axis=0)
    ntb = T // _TB
    tgt3 = tgt.reshape(ntb, _TB, B)
    grid_spec = pltpu.PrefetchScalarGridSpec(
        num_scalar_prefetch=1,
        grid=(ntb,),
        in_specs=[
            pl.BlockSpec((_TB, B, V), lambda i, lens: (i, 0, 0)),
            pl.BlockSpec((1, _TB, B), lambda i, lens: (i, 0, 0)),
        ],
        out_specs=pl.BlockSpec(memory_space=pltpu.SMEM),
        scratch_shapes=[
            pltpu.SMEM((1,), jnp.float32),
            pltpu.SMEM((1,), jnp.float32),
        ],
    )
    acc = pl.pallas_call(
        _acc_kernel,
        grid_spec=grid_spec,
        out_shape=jax.ShapeDtypeStruct((1,), jnp.float32),
        compiler_params=pltpu.CompilerParams(
            dimension_semantics=("arbitrary",),
        ),
    )(tokens_lens, outputs, tgt3)
    return acc[0]


# speedup vs baseline: 2.5324x; 1.0710x over previous
import jax
import jax.numpy as jnp
from jax import lax
from jax.experimental import pallas as pl
from jax.experimental.pallas import tpu as pltpu

_TB = 256


def _acc_kernel(lens_ref, x_ref, tgt_ref, out_ref, c_ref, m_ref):
    i = pl.program_id(0)
    nsteps = pl.num_programs(0)
    x = x_ref[...]
    TB, B, V = x.shape
    rowmax = jnp.max(x, axis=-1, keepdims=True)
    idx = lax.broadcasted_iota(jnp.int32, x.shape, 2)
    pred = jnp.min(jnp.where(x == rowmax, idx, V), axis=-1)
    tgt = tgt_ref[0]
    t_idx = lax.broadcasted_iota(jnp.int32, (TB, B), 0) + i * TB
    b_idx = lax.broadcasted_iota(jnp.int32, (TB, B), 1)
    lens_v = jnp.zeros((TB, B), jnp.int32)
    for b in range(B):
        lens_v = jnp.where(b_idx == b, lens_ref[b] + 1, lens_v)
    mask = t_idx < lens_v
    c_part = jnp.sum(jnp.where(mask & (pred == tgt), 1.0, 0.0))
    m_part = jnp.sum(jnp.where(mask, 1.0, 0.0))

    @pl.when(i == 0)
    def _init():
        c_ref[0] = 0.0
        m_ref[0] = 0.0

    c_ref[0] += c_part
    m_ref[0] += m_part

    @pl.when(i == nsteps - 1)
    def _fin():
        out_ref[0] = c_ref[0] / m_ref[0]


def kernel(outputs, tokens, tokens_lens):
    T, B, V = outputs.shape
    tgt = jnp.concatenate([tokens[1:], tokens[-1:]], axis=0)
    ntb = T // _TB
    tgt3 = tgt.reshape(ntb, _TB, B)
    grid_spec = pltpu.PrefetchScalarGridSpec(
        num_scalar_prefetch=1,
        grid=(ntb,),
        in_specs=[
            pl.BlockSpec((_TB, B, V), lambda i, lens: (i, 0, 0)),
            pl.BlockSpec((1, _TB, B), lambda i, lens: (i, 0, 0)),
        ],
        out_specs=pl.BlockSpec(memory_space=pltpu.SMEM),
        scratch_shapes=[
            pltpu.SMEM((1,), jnp.float32),
            pltpu.SMEM((1,), jnp.float32),
        ],
    )
    acc = pl.pallas_call(
        _acc_kernel,
        grid_spec=grid_spec,
        out_shape=jax.ShapeDtypeStruct((1,), jnp.float32),
        compiler_params=pltpu.CompilerParams(
            dimension_semantics=("arbitrary",),
        ),
    )(tokens_lens, outputs, tgt3)
    return acc[0]
